# baseline (device time: 62228 ns/iter reference)
import jax
import jax.numpy as jnp
from jax import lax
from jax.experimental import pallas as pl
from jax.experimental.pallas import tpu as pltpu

N_DEV = 16
B, SQ, D = 4, 256, 1024
HQ_LOC, DH = 8, 128
KV_COLS = 2 * DH
T = B * SQ
SCALE = 0.08838834764831843
WIRE = jnp.float32


def _fused_body(x_ref, wq_ref, wo_ref, wk_ref, wv_ref, out_ref,
                wk_sl, wv_sl, q_ref, k_ref, v_ref, att_ref,
                c1_ref, c2_ref, c3_ref, c4a_ref, c4b_ref, c5_ref, sb_ref,
                dma_sems, send_sems, recv_sems):
    i = lax.axis_index("i")
    Q = i % 4
    Pz = i // 4
    px = jnp.bitwise_and(jnp.bitwise_xor(Q, Q // 2), 1)
    py = Q // 2
    xp = Pz * 4 + jnp.bitwise_xor(Q, 1)
    yp = Pz * 4 + jnp.bitwise_xor(Q, 3)
    pz1 = jnp.bitwise_xor(i, 4)
    pz2 = jnp.bitwise_xor(i, 8)
    t = Pz % 2
    u = Pz // 2

    cp_k = pltpu.make_async_copy(
        wk_ref.at[:, pl.ds(i * KV_COLS, KV_COLS)], wk_sl, dma_sems.at[0])
    cp_v = pltpu.make_async_copy(
        wv_ref.at[:, pl.ds(i * KV_COLS, KV_COLS)], wv_sl, dma_sems.at[1])
    cp_k.start()
    cp_v.start()

    x2 = x_ref[...].reshape(T, D)

    def qkv_group(g):
        r0 = 512 * g
        xg = x2[r0:r0 + 512, :]
        q_ref[r0:r0 + 512, :] = jnp.dot(
            xg, wq_ref[...], preferred_element_type=jnp.float32)
        k_ref[r0:r0 + 512, :] = jnp.dot(
            xg, wk_sl[...], preferred_element_type=jnp.float32)
        v_ref[r0:r0 + 512, :] = jnp.dot(
            xg, wv_sl[...], preferred_element_type=jnp.float32)

    def attn_batch(b):
        r0 = b * SQ
        for h in range(HQ_LOC):
            g = h // 4
            qh = q_ref[r0:r0 + SQ, h * DH:(h + 1) * DH]
            kg = k_ref[r0:r0 + SQ, g * DH:(g + 1) * DH]
            vg = v_ref[r0:r0 + SQ, g * DH:(g + 1) * DH]
            s = lax.dot_general(
                qh, kg, (((1,), (1,)), ((), ())),
                preferred_element_type=jnp.float32) * SCALE
            m = jnp.max(s, axis=1, keepdims=True)
            p = jnp.exp(s - m)
            l = jnp.sum(p, axis=1, keepdims=True)
            o = jnp.dot(p, vg, preferred_element_type=jnp.float32) / l
            att_ref[r0:r0 + SQ, h * DH:(h + 1) * DH] = o

    def wo_group(g):
        r0 = 512 * g
        out_ref[r0:r0 + 512, :] = jnp.dot(
            att_ref[r0:r0 + 512, :], wo_ref[...],
            preferred_element_type=jnp.float32)

    sends = []
    pending = []
    cursor = [0]

    def xfer(src_rows, n_rows, dst, sem_idx, dev, packed_src=None):
        if packed_src is None:
            s0 = cursor[0]
            h = n_rows // 2
            cursor[0] += h
            sb_ref[s0:s0 + h, :] = pltpu.bitcast(
                out_ref[pl.ds(src_rows, n_rows), :].astype(jnp.bfloat16),
                jnp.float32)
            packed_src = sb_ref.at[pl.ds(s0, h), :]
        rdma = pltpu.make_async_remote_copy(
            src_ref=packed_src,
            dst_ref=dst,
            send_sem=send_sems.at[sem_idx],
            recv_sem=recv_sems.at[sem_idx],
            device_id=(dev,),
            device_id_type=pl.DeviceIdType.MESH,
        )
        rdma.start()
        sends.append(rdma)
        return rdma

    def step_done(keep=2):
        pending.append(list(sends))
        sends.clear()
        while len(pending) > keep:
            for r in pending.pop(0):
                r.wait_send()

    def unpack(buf):
        return pltpu.bitcast(buf, jnp.bfloat16).astype(jnp.float32)

    def acc(rows, n_rows, buf):
        out_ref[pl.ds(rows, n_rows), :] = (
            out_ref[pl.ds(rows, n_rows), :] + unpack(buf))

    def store(rows, n_rows, buf):
        out_ref[pl.ds(rows, n_rows), :] = unpack(buf)

    def p1_start(g, s):
        base = 512 * g
        off = 16 * g + 2 * s
        if s == 0:
            rT = xfer(base + 128 * (1 - px), 128,
                      c1_ref.at[4 * g, pl.ds(0, 64)], off, xp)
            rB = xfer(base + 256 + 128 * (1 - py), 128,
                      c1_ref.at[4 * g + 1, pl.ds(0, 64)], off + 1, yp)
        else:
            rT = xfer(base + 128 * px + 64 * (1 - py), 64,
                      c1_ref.at[4 * g + 2, pl.ds(0, 32)], off, yp)
            rB = xfer(base + 256 + 128 * py + 64 * (1 - px), 64,
                      c1_ref.at[4 * g + 3, pl.ds(0, 32)], off + 1, xp)
        return rT, rB

    def p1_finish(g, s, rT, rB):
        base = 512 * g
        if s == 0:
            rT.wait_recv()
            acc(base + 128 * px, 128, c1_ref[4 * g, :64])
            rB.wait_recv()
            acc(base + 256 + 128 * py, 128, c1_ref[4 * g + 1, :64])
        else:
            rT.wait_recv()
            acc(base + 128 * px + 64 * py, 64, c1_ref[4 * g + 2, :32])
            rB.wait_recv()
            acc(base + 256 + 128 * py + 64 * px, 64,
                c1_ref[4 * g + 3, :32])

    def baseR(g):
        return 512 * g + 128 * px + 64 * py

    def baseL(g):
        return 512 * g + 256 + 128 * py + 64 * px

    def p2_start(g, s):
        off = 16 * g + 4 + 2 * s
        if s == 0:
            rT = xfer(baseR(g) + 32 * (1 - t), 32, c2_ref.at[2 * g],
                      off, pz1)
            rB = xfer(baseL(g) + 32 * (1 - t), 32, c2_ref.at[2 * g + 1],
                      off + 1, pz1)
        else:
            rT = xfer(baseR(g) + 32 * t + 16 * (1 - u), 16,
                      c3_ref.at[2 * g], off, pz2)
            rB = xfer(baseL(g) + 32 * t + 16 * (1 - u), 16,
                      c3_ref.at[2 * g + 1], off + 1, pz2)
        return rT, rB

    def p2_finish(g, s, rT, rB):
        if s == 0:
            rT.wait_recv()
            acc(baseR(g) + 32 * t, 32, c2_ref[2 * g])
            rB.wait_recv()
            acc(baseL(g) + 32 * t, 32, c2_ref[2 * g + 1])
        else:
            rT.wait_recv()
            acc(baseR(g) + 32 * t + 16 * u, 16, c3_ref[2 * g])
            rB.wait_recv()
            acc(baseL(g) + 32 * t + 16 * u, 16, c3_ref[2 * g + 1])

    def p3a_start(g, s):
        off = 16 * g + 8 + 2 * s
        if s == 0:
            qT = baseR(g) + 32 * t + 16 * u
            qB = baseL(g) + 32 * t + 16 * u
            rT = xfer(qT, 16, c4a_ref.at[2 * g], off, pz2)
            rB = xfer(qB, 16, c4a_ref.at[2 * g + 1], off + 1, pz2)
        else:
            hT = baseR(g) + 32 * t
            hB = baseL(g) + 32 * t
            rT = xfer(hT, 32, c4b_ref.at[2 * g], off, pz1)
            rB = xfer(hB, 32, c4b_ref.at[2 * g + 1], off + 1, pz1)
        return rT, rB

    def p3a_finish(g, s, rT, rB):
        if s == 0:
            rT.wait_recv()
            store(baseR(g) + 32 * t + 16 * (1 - u), 16, c4a_ref[2 * g])
            rB.wait_recv()
            store(baseL(g) + 32 * t + 16 * (1 - u), 16, c4a_ref[2 * g + 1])
        else:
            rT.wait_recv()
            store(baseR(g) + 32 * (1 - t), 32, c4b_ref[2 * g])
            rB.wait_recv()
            store(baseL(g) + 32 * (1 - t), 32, c4b_ref[2 * g + 1])

    def p3b_start(g, s):
        base = 512 * g
        off = 16 * g + 12 + 2 * s
        if s == 0:
            rT = xfer(baseR(g), 64, c5_ref.at[4 * g, pl.ds(0, 32)],
                      off, yp)
            rB = xfer(baseL(g), 64, c5_ref.at[4 * g + 1, pl.ds(0, 32)],
                      off + 1, xp)
        else:
            rT = xfer(base + 128 * px, 128,
                      c5_ref.at[4 * g + 2, pl.ds(0, 64)], off, xp)
            rB = xfer(base + 256 + 128 * py, 128,
                      c5_ref.at[4 * g + 3, pl.ds(0, 64)], off + 1, yp)
        return rT, rB

    def p3b_finish(g, s, rT, rB):
        base = 512 * g
        if s == 0:
            rT.wait_recv()
            store(base + 128 * px + 64 * (1 - py), 64,
                  c5_ref[4 * g, :32])
            rB.wait_recv()
            store(base + 256 + 128 * py + 64 * (1 - px), 64,
                  c5_ref[4 * g + 1, :32])
        else:
            rT.wait_recv()
            store(base + 128 * (1 - px), 128, c5_ref[4 * g + 2, :64])
            rB.wait_recv()
            store(base + 256 + 128 * (1 - py), 128, c5_ref[4 * g + 3, :64])

    cp_k.wait()
    cp_v.wait()
    qkv_group(0)
    attn_batch(0)
    attn_batch(1)
    wo_group(0)

    barrier_sem = pltpu.get_barrier_semaphore()
    for nbr in (xp, yp, pz1, pz2):
        pl.semaphore_signal(
            barrier_sem, inc=1,
            device_id=(nbr,), device_id_type=pl.DeviceIdType.MESH)
    pl.semaphore_wait(barrier_sem, 4)

    r = p1_start(0, 0)
    qkv_group(1)
    p1_finish(0, 0, *r)
    step_done()
    r = p1_start(0, 1)
    attn_batch(2)
    attn_batch(3)
    p1_finish(0, 1, *r)
    step_done()

    a = p2_start(0, 0)
    wo_group(1)
    b = p1_start(1, 0)
    p2_finish(0, 0, *a); p1_finish(1, 0, *b)
    step_done()

    a = p2_start(0, 1); b = p1_start(1, 1)
    p2_finish(0, 1, *a); p1_finish(1, 1, *b)
    step_done()

    a = p3a_start(0, 0); b = p2_start(1, 0)
    p3a_finish(0, 0, *a); p2_finish(1, 0, *b)
    step_done()

    a = p3a_start(0, 1); b = p2_start(1, 1)
    p3a_finish(0, 1, *a); p2_finish(1, 1, *b)
    step_done()

    a = p3b_start(0, 0); b = p3a_start(1, 0)
    p3b_finish(0, 0, *a); p3a_finish(1, 0, *b)
    step_done()

    a = p3b_start(0, 1); b = p3a_start(1, 1)
    p3b_finish(0, 1, *a); p3a_finish(1, 1, *b)
    step_done()

    b = p3b_start(1, 0)
    p3b_finish(1, 0, *b)
    step_done()
    b = p3b_start(1, 1)
    p3b_finish(1, 1, *b)
    step_done(keep=0)


def kernel(x, Wq, Wo, Wk, Wv):
    reduced = pl.pallas_call(
        _fused_body,
        out_shape=jax.ShapeDtypeStruct((T, D), jnp.float32),
        in_specs=[
            pl.BlockSpec(memory_space=pltpu.VMEM),
            pl.BlockSpec(memory_space=pltpu.VMEM),
            pl.BlockSpec(memory_space=pltpu.VMEM),
            pl.BlockSpec(memory_space=pltpu.MemorySpace.HBM),
            pl.BlockSpec(memory_space=pltpu.MemorySpace.HBM),
        ],
        out_specs=pl.BlockSpec(memory_space=pltpu.VMEM),
        scratch_shapes=[
            pltpu.VMEM((D, KV_COLS), jnp.float32),
            pltpu.VMEM((D, KV_COLS), jnp.float32),
            pltpu.VMEM((T, D), jnp.float32),
            pltpu.VMEM((T, KV_COLS), jnp.float32),
            pltpu.VMEM((T, KV_COLS), jnp.float32),
            pltpu.VMEM((T, D), jnp.float32),
            pltpu.VMEM((8, 64, D), WIRE),
            pltpu.VMEM((4, 16, D), WIRE),
            pltpu.VMEM((4, 8, D), WIRE),
            pltpu.VMEM((4, 8, D), WIRE),
            pltpu.VMEM((4, 16, D), WIRE),
            pltpu.VMEM((8, 64, D), WIRE),
            pltpu.VMEM((960, D), WIRE),
            pltpu.SemaphoreType.DMA((2,)),
            pltpu.SemaphoreType.DMA((32,)),
            pltpu.SemaphoreType.DMA((32,)),
        ],
        compiler_params=pltpu.CompilerParams(collective_id=0),
    )(x, Wq, Wo, Wk, Wv)

    return reduced.reshape(B, SQ, D)


# device time: 59913 ns/iter; 1.0386x vs baseline; 1.0386x over previous
import jax
import jax.numpy as jnp
from jax import lax
from jax.experimental import pallas as pl
from jax.experimental.pallas import tpu as pltpu

N_DEV = 16
B, SQ, D = 4, 256, 1024
HQ_LOC, DH = 8, 128
KV_COLS = 2 * DH
T = B * SQ
SCALE = 0.08838834764831843
WIRE = jnp.float32


def _fused_body(x_ref, wq_ref, wo_ref, wk_ref, wv_ref, out_ref,
                wk_sl, wv_sl, q_ref, k_ref, v_ref, att_ref,
                c1_ref, c2_ref, c3_ref, c4a_ref, c4b_ref, c5_ref, sb_ref,
                dma_sems, send_sems, recv_sems):
    i = lax.axis_index("i")
    Q = i % 4
    Pz = i // 4
    px = jnp.bitwise_and(jnp.bitwise_xor(Q, Q // 2), 1)
    py = Q // 2
    xp = Pz * 4 + jnp.bitwise_xor(Q, 1)
    yp = Pz * 4 + jnp.bitwise_xor(Q, 3)
    pz1 = jnp.bitwise_xor(i, 4)
    pz2 = jnp.bitwise_xor(i, 8)
    t = Pz % 2
    u = Pz // 2

    cp_k = pltpu.make_async_copy(
        wk_ref.at[:, pl.ds(i * KV_COLS, KV_COLS)], wk_sl, dma_sems.at[0])
    cp_v = pltpu.make_async_copy(
        wv_ref.at[:, pl.ds(i * KV_COLS, KV_COLS)], wv_sl, dma_sems.at[1])
    cp_k.start()
    cp_v.start()

    x2 = x_ref[...].reshape(T, D)

    def qkv_group(g):
        r0 = 512 * g
        xg = x2[r0:r0 + 512, :]
        q_ref[r0:r0 + 512, :] = jnp.dot(
            xg, wq_ref[...], preferred_element_type=jnp.float32)
        k_ref[r0:r0 + 512, :] = jnp.dot(
            xg, wk_sl[...], preferred_element_type=jnp.float32)
        v_ref[r0:r0 + 512, :] = jnp.dot(
            xg, wv_sl[...], preferred_element_type=jnp.float32)

    def attn_batch(b):
        r0 = b * SQ
        for h in range(HQ_LOC):
            g = h // 4
            qh = q_ref[r0:r0 + SQ, h * DH:(h + 1) * DH]
            kg = k_ref[r0:r0 + SQ, g * DH:(g + 1) * DH]
            vg = v_ref[r0:r0 + SQ, g * DH:(g + 1) * DH]
            s = lax.dot_general(
                qh, kg, (((1,), (1,)), ((), ())),
                preferred_element_type=jnp.float32) * SCALE
            m = jnp.max(s, axis=1, keepdims=True)
            p = jnp.exp(s - m)
            l = jnp.sum(p, axis=1, keepdims=True)
            o = jnp.dot(p, vg, preferred_element_type=jnp.float32) / l
            att_ref[r0:r0 + SQ, h * DH:(h + 1) * DH] = o

    def wo_group(g):
        r0 = 512 * g
        out_ref[r0:r0 + 512, :] = jnp.dot(
            att_ref[r0:r0 + 512, :], wo_ref[...],
            preferred_element_type=jnp.float32)

    sends = []
    pending = []
    cursor = [0]

    def xfer(src_rows, n_rows, dst, sem_idx, dev, packed_src=None):
        if packed_src is None:
            s0 = cursor[0]
            h = n_rows // 2
            cursor[0] += h
            sb_ref[s0:s0 + h, :] = pltpu.bitcast(
                out_ref[pl.ds(src_rows, n_rows), :].astype(jnp.bfloat16),
                jnp.float32)
            packed_src = sb_ref.at[pl.ds(s0, h), :]
        rdma = pltpu.make_async_remote_copy(
            src_ref=packed_src,
            dst_ref=dst,
            send_sem=send_sems.at[sem_idx],
            recv_sem=recv_sems.at[sem_idx],
            device_id=(dev,),
            device_id_type=pl.DeviceIdType.MESH,
        )
        rdma.start()
        sends.append(rdma)
        return rdma

    def step_done(keep=2):
        pending.append(list(sends))
        sends.clear()
        while len(pending) > keep:
            for r in pending.pop(0):
                r.wait_send()

    def unpack(buf):
        return pltpu.bitcast(buf, jnp.bfloat16).astype(jnp.float32)

    def acc(rows, n_rows, buf):
        out_ref[pl.ds(rows, n_rows), :] = (
            out_ref[pl.ds(rows, n_rows), :] + unpack(buf))

    def store(rows, n_rows, buf):
        out_ref[pl.ds(rows, n_rows), :] = unpack(buf)

    def p1_start(g, s):
        base = 512 * g
        off = 16 * g + 2 * s
        if s == 0:
            rT = xfer(base + 128 * (1 - px), 128,
                      c1_ref.at[4 * g, pl.ds(0, 64)], off, xp)
            rB = xfer(base + 256 + 128 * (1 - py), 128,
                      c1_ref.at[4 * g + 1, pl.ds(0, 64)], off + 1, yp)
        else:
            rT = xfer(base + 128 * px + 64 * (1 - py), 64,
                      c1_ref.at[4 * g + 2, pl.ds(0, 32)], off, yp)
            rB = xfer(base + 256 + 128 * py + 64 * (1 - px), 64,
                      c1_ref.at[4 * g + 3, pl.ds(0, 32)], off + 1, xp)
        return rT, rB

    def p1_finish(g, s, rT, rB):
        base = 512 * g
        if s == 0:
            rT.wait_recv()
            acc(base + 128 * px, 128, c1_ref[4 * g, :64])
            rB.wait_recv()
            acc(base + 256 + 128 * py, 128, c1_ref[4 * g + 1, :64])
        else:
            rT.wait_recv()
            acc(base + 128 * px + 64 * py, 64, c1_ref[4 * g + 2, :32])
            rB.wait_recv()
            acc(base + 256 + 128 * py + 64 * px, 64,
                c1_ref[4 * g + 3, :32])

    def baseR(g):
        return 512 * g + 128 * px + 64 * py

    def baseL(g):
        return 512 * g + 256 + 128 * py + 64 * px

    def p2_start(g, s):
        off = 16 * g + 4 + 2 * s
        if s == 0:
            rT = xfer(baseR(g) + 32 * (1 - t), 32, c2_ref.at[2 * g],
                      off, pz1)
            rB = xfer(baseL(g) + 32 * (1 - t), 32, c2_ref.at[2 * g + 1],
                      off + 1, pz1)
        else:
            rT = xfer(baseR(g) + 32 * t + 16 * (1 - u), 16,
                      c3_ref.at[2 * g], off, pz2)
            rB = xfer(baseL(g) + 32 * t + 16 * (1 - u), 16,
                      c3_ref.at[2 * g + 1], off + 1, pz2)
        return rT, rB

    def p2_finish(g, s, rT, rB):
        if s == 0:
            rT.wait_recv()
            acc(baseR(g) + 32 * t, 32, c2_ref[2 * g])
            rB.wait_recv()
            acc(baseL(g) + 32 * t, 32, c2_ref[2 * g + 1])
        else:
            rT.wait_recv()
            acc(baseR(g) + 32 * t + 16 * u, 16, c3_ref[2 * g])
            rB.wait_recv()
            acc(baseL(g) + 32 * t + 16 * u, 16, c3_ref[2 * g + 1])

    def p3a_start(g, s):
        off = 16 * g + 8 + 2 * s
        if s == 0:
            qT = baseR(g) + 32 * t + 16 * u
            qB = baseL(g) + 32 * t + 16 * u
            rT = xfer(qT, 16, c4a_ref.at[2 * g], off, pz2)
            rB = xfer(qB, 16, c4a_ref.at[2 * g + 1], off + 1, pz2)
        else:
            hT = baseR(g) + 32 * t
            hB = baseL(g) + 32 * t
            rT = xfer(hT, 32, c4b_ref.at[2 * g], off, pz1)
            rB = xfer(hB, 32, c4b_ref.at[2 * g + 1], off + 1, pz1)
        return rT, rB

    def p3a_finish(g, s, rT, rB):
        if s == 0:
            rT.wait_recv()
            store(baseR(g) + 32 * t + 16 * (1 - u), 16, c4a_ref[2 * g])
            rB.wait_recv()
            store(baseL(g) + 32 * t + 16 * (1 - u), 16, c4a_ref[2 * g + 1])
        else:
            rT.wait_recv()
            store(baseR(g) + 32 * (1 - t), 32, c4b_ref[2 * g])
            rB.wait_recv()
            store(baseL(g) + 32 * (1 - t), 32, c4b_ref[2 * g + 1])

    def p3b_start(g, s):
        base = 512 * g
        off = 16 * g + 12 + 2 * s
        if s == 0:
            rT = xfer(baseR(g), 64, c5_ref.at[4 * g, pl.ds(0, 32)],
                      off, yp)
            rB = xfer(baseL(g), 64, c5_ref.at[4 * g + 1, pl.ds(0, 32)],
                      off + 1, xp)
        else:
            rT = xfer(base + 128 * px, 128,
                      c5_ref.at[4 * g + 2, pl.ds(0, 64)], off, xp)
            rB = xfer(base + 256 + 128 * py, 128,
                      c5_ref.at[4 * g + 3, pl.ds(0, 64)], off + 1, yp)
        return rT, rB

    def p3b_finish(g, s, rT, rB):
        base = 512 * g
        if s == 0:
            rT.wait_recv()
            store(base + 128 * px + 64 * (1 - py), 64,
                  c5_ref[4 * g, :32])
            rB.wait_recv()
            store(base + 256 + 128 * py + 64 * (1 - px), 64,
                  c5_ref[4 * g + 1, :32])
        else:
            rT.wait_recv()
            store(base + 128 * (1 - px), 128, c5_ref[4 * g + 2, :64])
            rB.wait_recv()
            store(base + 256 + 128 * (1 - py), 128, c5_ref[4 * g + 3, :64])

    cp_k.wait()
    cp_v.wait()
    qkv_group(0)
    attn_batch(0)
    attn_batch(1)
    wo_group(0)

    barrier_sem = pltpu.get_barrier_semaphore()
    for nbr in (xp, yp, pz1, pz2):
        pl.semaphore_signal(
            barrier_sem, inc=1,
            device_id=(nbr,), device_id_type=pl.DeviceIdType.MESH)
    pl.semaphore_wait(barrier_sem, 4)

    r = p1_start(0, 0)
    qkv_group(1)
    attn_batch(2)
    p1_finish(0, 0, *r)
    step_done()
    r = p1_start(0, 1)
    attn_batch(3)
    wo_group(1)
    p1_finish(0, 1, *r)
    step_done()

    a = p2_start(0, 0); b = p1_start(1, 0)
    p2_finish(0, 0, *a); p1_finish(1, 0, *b)
    step_done()

    a = p2_start(0, 1); b = p1_start(1, 1)
    p2_finish(0, 1, *a); p1_finish(1, 1, *b)
    step_done()

    a = p3a_start(0, 0); b = p2_start(1, 0)
    p3a_finish(0, 0, *a); p2_finish(1, 0, *b)
    step_done()

    a = p3a_start(0, 1); b = p2_start(1, 1)
    p3a_finish(0, 1, *a); p2_finish(1, 1, *b)
    step_done()

    a = p3b_start(0, 0); b = p3a_start(1, 0)
    p3b_finish(0, 0, *a); p3a_finish(1, 0, *b)
    step_done()

    a = p3b_start(0, 1); b = p3a_start(1, 1)
    p3b_finish(0, 1, *a); p3a_finish(1, 1, *b)
    step_done()

    b = p3b_start(1, 0)
    p3b_finish(1, 0, *b)
    step_done()
    b = p3b_start(1, 1)
    p3b_finish(1, 1, *b)
    step_done(keep=0)


def kernel(x, Wq, Wo, Wk, Wv):
    reduced = pl.pallas_call(
        _fused_body,
        out_shape=jax.ShapeDtypeStruct((T, D), jnp.float32),
        in_specs=[
            pl.BlockSpec(memory_space=pltpu.VMEM),
            pl.BlockSpec(memory_space=pltpu.VMEM),
            pl.BlockSpec(memory_space=pltpu.VMEM),
            pl.BlockSpec(memory_space=pltpu.MemorySpace.HBM),
            pl.BlockSpec(memory_space=pltpu.MemorySpace.HBM),
        ],
        out_specs=pl.BlockSpec(memory_space=pltpu.VMEM),
        scratch_shapes=[
            pltpu.VMEM((D, KV_COLS), jnp.float32),
            pltpu.VMEM((D, KV_COLS), jnp.float32),
            pltpu.VMEM((T, D), jnp.float32),
            pltpu.VMEM((T, KV_COLS), jnp.float32),
            pltpu.VMEM((T, KV_COLS), jnp.float32),
            pltpu.VMEM((T, D), jnp.float32),
            pltpu.VMEM((8, 64, D), WIRE),
            pltpu.VMEM((4, 16, D), WIRE),
            pltpu.VMEM((4, 8, D), WIRE),
            pltpu.VMEM((4, 8, D), WIRE),
            pltpu.VMEM((4, 16, D), WIRE),
            pltpu.VMEM((8, 64, D), WIRE),
            pltpu.VMEM((960, D), WIRE),
            pltpu.SemaphoreType.DMA((2,)),
            pltpu.SemaphoreType.DMA((32,)),
            pltpu.SemaphoreType.DMA((32,)),
        ],
        compiler_params=pltpu.CompilerParams(collective_id=0),
    )(x, Wq, Wo, Wk, Wv)

    return reduced.reshape(B, SQ, D)
